# baseline (speedup 1.0000x reference)
import jax
import jax.numpy as jnp
from jax import lax
from jax.experimental import pallas as pl
from jax.experimental.pallas import tpu as pltpu
from jax.experimental.pallas import tpu_sc as plsc

NC = 2
NS = 16
K = 125


def _deg_body(edge_hbm, ones_hbm, zeros_hbm, out_hbm, dacc, idx2, ones_v):
    c = lax.axis_index("c")
    s = lax.axis_index("s")
    t = c * NS + s
    slab = dacc.shape[0] // NS
    cpt = edge_hbm.shape[2]
    pltpu.sync_copy(zeros_hbm, dacc.at[pl.ds(s * slab, slab)])
    pltpu.sync_copy(ones_hbm, ones_v)
    plsc.subcore_barrier()

    def body(j, carry):
        pltpu.sync_copy(edge_hbm.at[1, t, j], idx2)
        pltpu.sync_copy(ones_v, dacc.at[idx2.at[0]], add=True)
        return carry

    lax.fori_loop(0, cpt, body, 0)
    plsc.subcore_barrier()
    pltpu.sync_copy(dacc.at[pl.ds(s * slab, slab)],
                    out_hbm.at[c, pl.ds(s * slab, slab)])


def _prop_body(hs_hbm, edge_hbm, zeros_hbm, out_hbm, acc, srcs, dsts, rows):
    c = lax.axis_index("c")
    s = lax.axis_index("s")
    t = c * NS + s
    slab = acc.shape[0] // NS
    pltpu.sync_copy(zeros_hbm, acc.at[pl.ds(s * slab, slab)])
    pltpu.sync_copy(edge_hbm.at[0, t], srcs)
    pltpu.sync_copy(edge_hbm.at[1, t], dsts)
    plsc.subcore_barrier()

    def body(j, carry):
        pltpu.sync_copy(hs_hbm.at[srcs.at[j]], rows)
        pltpu.sync_copy(rows, acc.at[dsts.at[j]], add=True)
        return carry

    lax.fori_loop(0, srcs.shape[0], body, 0)
    plsc.subcore_barrier()
    pltpu.sync_copy(acc.at[pl.ds(s * slab, slab)],
                    out_hbm.at[c, pl.ds(s * slab, slab)])


def _dinv_b(degp_ref, n, d):
    dp = degp_ref[...]
    deg = dp[0, :n, :1] + dp[1, :n, :1] + 1.0
    return jnp.broadcast_to(lax.rsqrt(deg), (n, d))


def _tc1_body(x_ref, w1_ref, degp_ref, hs1_ref, dinv_ref):
    n, d = hs1_ref.shape
    dinv_b = _dinv_b(degp_ref, n, d)
    h = jnp.dot(x_ref[...], w1_ref[...],
                preferred_element_type=jnp.float32)
    hs1_ref[...] = h * dinv_b
    dinv_ref[...] = dinv_b


def _tc_mid_body(p_ref, hs1_ref, dinv_ref, b1_ref, w2_ref, wlin_ref,
                 hs2_ref, acc_ref):
    n, d = hs1_ref.shape
    dinv = dinv_ref[...]
    p = p_ref[...]
    x1 = jnp.maximum(
        dinv * (p[0, :n] + p[1, :n] + hs1_ref[...]) + b1_ref[...], 0.0)
    hs2_ref[...] = jnp.dot(x1, w2_ref[...],
                           preferred_element_type=jnp.float32) * dinv
    acc_ref[...] = jnp.dot(x1, wlin_ref[:d, :],
                           preferred_element_type=jnp.float32)


def _tc_fin_body(q_ref, hs2_ref, dinv_ref, b2_ref, acc_ref, wlin_ref,
                 blin_ref, out_ref):
    n, d = hs2_ref.shape
    dinv = dinv_ref[...]
    q = q_ref[...]
    x2 = jnp.maximum(
        dinv * (q[0, :n] + q[1, :n] + hs2_ref[...]) + b2_ref[...], 0.0)
    out_ref[...] = acc_ref[...] + jnp.dot(
        x2, wlin_ref[d:, :], preferred_element_type=jnp.float32) + blin_ref[...]


def kernel(x, edge_index, W1, b1, W2, b2, Wlin, blin):
    n, d_in = x.shape
    e = edge_index.shape[1]
    d_hid = W1.shape[1]
    d_out = Wlin.shape[1]
    assert e % (K * NC * NS) == 0
    npad = ((n + 8 * NS - 1) // (8 * NS)) * (8 * NS)
    slab = npad // NS
    cpt = e // K // (NC * NS)
    assert cpt % 16 == 0

    edge_r = edge_index.reshape(2, NC * NS, cpt, K)
    edge_d5 = edge_index.reshape(2, NC * NS, cpt, 1, K)
    zeros_p = jnp.zeros((slab, 128), jnp.float32)
    zeros_d = zeros_p
    ones_k = jnp.ones((K, 128), jnp.float32)

    mesh = plsc.VectorSubcoreMesh(core_axis_name="c", subcore_axis_name="s")

    deg_call = pl.kernel(
        _deg_body,
        out_type=jax.ShapeDtypeStruct((NC, npad, 128), jnp.float32),
        mesh=mesh,
        scratch_types=[
            pltpu.VMEM_SHARED((npad, 128), jnp.float32),
            pltpu.VMEM((1, K), jnp.int32),
            pltpu.VMEM((K, 128), jnp.float32),
        ],
    )
    prop_call = pl.kernel(
        _prop_body,
        out_type=jax.ShapeDtypeStruct((NC, npad, d_hid), jnp.float32),
        mesh=mesh,
        scratch_types=[
            pltpu.VMEM_SHARED((npad, d_hid), jnp.float32),
            pltpu.VMEM((cpt, K), jnp.int32),
            pltpu.VMEM((cpt, K), jnp.int32),
            pltpu.VMEM((K, d_hid), jnp.float32),
        ],
    )

    degp = deg_call(edge_d5, ones_k, zeros_d)

    hs1, dinv_b = pl.pallas_call(
        _tc1_body,
        out_shape=[
            jax.ShapeDtypeStruct((n, d_hid), jnp.float32),
            jax.ShapeDtypeStruct((n, d_hid), jnp.float32),
        ],
    )(x, W1, degp)

    p_part = prop_call(hs1, edge_r, zeros_p)

    hs2, acc = pl.pallas_call(
        _tc_mid_body,
        out_shape=[
            jax.ShapeDtypeStruct((n, d_hid), jnp.float32),
            jax.ShapeDtypeStruct((n, d_out), jnp.float32),
        ],
    )(p_part, hs1, dinv_b, b1, W2, Wlin)

    q_part = prop_call(hs2, edge_r, zeros_p)

    out = pl.pallas_call(
        _tc_fin_body,
        out_shape=jax.ShapeDtypeStruct((n, d_out), jnp.float32),
    )(q_part, hs2, dinv_b, b2, acc, Wlin, blin)

    return out


# speedup vs baseline: 1.3380x; 1.3380x over previous
import jax
import jax.numpy as jnp
from jax import lax
from jax.experimental import pallas as pl
from jax.experimental.pallas import tpu as pltpu
from jax.experimental.pallas import tpu_sc as plsc

NC = 2
NS = 16
K = 125


def _deg_body(edge_hbm, ones_hbm, zeros_hbm, out_hbm, dacc, idx2, ones_v):
    c = lax.axis_index("c")
    s = lax.axis_index("s")
    t = c * NS + s
    slab = dacc.shape[0] // NS
    cpt = edge_hbm.shape[2]
    pltpu.sync_copy(zeros_hbm, dacc.at[pl.ds(s * slab, slab)])
    pltpu.sync_copy(ones_hbm, ones_v)
    plsc.subcore_barrier()

    def body(j, carry):
        pltpu.sync_copy(edge_hbm.at[1, t, j], idx2)
        pltpu.sync_copy(ones_v, dacc.at[idx2.at[0]], add=True)
        return carry

    lax.fori_loop(0, cpt, body, 0)
    plsc.subcore_barrier()
    pltpu.sync_copy(dacc.at[pl.ds(s * slab, slab)],
                    out_hbm.at[c, pl.ds(s * slab, slab)])


def _prop_body(hs_hbm, edge_hbm, zeros_hbm, out_hbm, acc, srcs, dsts,
               rows0, rows1, sem0, sem1):
    c = lax.axis_index("c")
    s = lax.axis_index("s")
    t = c * NS + s
    slab = acc.shape[0] // NS
    nhalf = edge_hbm.shape[2]
    half = srcs.shape[0]
    pltpu.sync_copy(zeros_hbm, acc.at[pl.ds(s * slab, slab)])
    plsc.subcore_barrier()

    for h in range(nhalf):
        pltpu.sync_copy(edge_hbm.at[0, t, h], srcs)
        pltpu.sync_copy(edge_hbm.at[1, t, h], dsts)
        pltpu.async_copy(hs_hbm.at[srcs.at[0]], rows0, sem0)

        def body(g, carry):
            j0 = g * 2
            j1 = j0 + 1
            pltpu.async_copy(hs_hbm.at[srcs.at[j1]], rows1, sem1)
            pltpu.make_async_copy(hs_hbm.at[srcs.at[j0]], rows0, sem0).wait()
            pltpu.sync_copy(rows0, acc.at[dsts.at[j0]], add=True)

            @pl.when(j0 + 2 < half)
            def _():
                pltpu.async_copy(hs_hbm.at[srcs.at[j0 + 2]], rows0, sem0)

            pltpu.make_async_copy(hs_hbm.at[srcs.at[j1]], rows1, sem1).wait()
            pltpu.sync_copy(rows1, acc.at[dsts.at[j1]], add=True)
            return carry

        lax.fori_loop(0, half // 2, body, 0)
    plsc.subcore_barrier()
    pltpu.sync_copy(acc.at[pl.ds(s * slab, slab)],
                    out_hbm.at[c, pl.ds(s * slab, slab)])


def _dinv_b(degp_ref, n, d):
    dp = degp_ref[...]
    deg = dp[0, :n, :1] + dp[1, :n, :1] + 1.0
    return jnp.broadcast_to(lax.rsqrt(deg), (n, d))


def _tc1_body(x_ref, w1_ref, degp_ref, hs1_ref, dinv_ref):
    n, d = hs1_ref.shape
    dinv_b = _dinv_b(degp_ref, n, d)
    h = jnp.dot(x_ref[...], w1_ref[...],
                preferred_element_type=jnp.float32)
    hs1_ref[...] = h * dinv_b
    dinv_ref[...] = dinv_b


def _tc_mid_body(p_ref, hs1_ref, dinv_ref, b1_ref, w2_ref, wlin_ref,
                 hs2_ref, acc_ref):
    n, d = hs1_ref.shape
    dinv = dinv_ref[...]
    p = p_ref[...]
    x1 = jnp.maximum(
        dinv * (p[0, :n] + p[1, :n] + hs1_ref[...]) + b1_ref[...], 0.0)
    hs2_ref[...] = jnp.dot(x1, w2_ref[...],
                           preferred_element_type=jnp.float32) * dinv
    acc_ref[...] = jnp.dot(x1, wlin_ref[:d, :],
                           preferred_element_type=jnp.float32)


def _tc_fin_body(q_ref, hs2_ref, dinv_ref, b2_ref, acc_ref, wlin_ref,
                 blin_ref, out_ref):
    n, d = hs2_ref.shape
    dinv = dinv_ref[...]
    q = q_ref[...]
    x2 = jnp.maximum(
        dinv * (q[0, :n] + q[1, :n] + hs2_ref[...]) + b2_ref[...], 0.0)
    out_ref[...] = acc_ref[...] + jnp.dot(
        x2, wlin_ref[d:, :], preferred_element_type=jnp.float32) + blin_ref[...]


def kernel(x, edge_index, W1, b1, W2, b2, Wlin, blin):
    n, d_in = x.shape
    e = edge_index.shape[1]
    d_hid = W1.shape[1]
    d_out = Wlin.shape[1]
    assert e % (K * NC * NS) == 0
    npad = ((n + 8 * NS - 1) // (8 * NS)) * (8 * NS)
    slab = npad // NS
    cpt = e // K // (NC * NS)
    assert cpt % 16 == 0

    edge_p5 = edge_index.reshape(2, NC * NS, 2, cpt // 2, K)
    edge_d5 = edge_index.reshape(2, NC * NS, cpt, 1, K)
    zeros_p = jnp.zeros((slab, 128), jnp.float32)
    ones_k = jnp.ones((K, 128), jnp.float32)

    mesh = plsc.VectorSubcoreMesh(core_axis_name="c", subcore_axis_name="s")

    deg_call = pl.kernel(
        _deg_body,
        out_type=jax.ShapeDtypeStruct((NC, npad, 128), jnp.float32),
        mesh=mesh,
        scratch_types=[
            pltpu.VMEM_SHARED((npad, 128), jnp.float32),
            pltpu.VMEM((1, K), jnp.int32),
            pltpu.VMEM((K, 128), jnp.float32),
        ],
    )
    prop_call = pl.kernel(
        _prop_body,
        out_type=jax.ShapeDtypeStruct((NC, npad, d_hid), jnp.float32),
        mesh=mesh,
        scratch_types=[
            pltpu.VMEM_SHARED((npad, d_hid), jnp.float32),
            pltpu.VMEM((cpt // 2, K), jnp.int32),
            pltpu.VMEM((cpt // 2, K), jnp.int32),
            pltpu.VMEM((K, d_hid), jnp.float32),
            pltpu.VMEM((K, d_hid), jnp.float32),
            pltpu.SemaphoreType.DMA,
            pltpu.SemaphoreType.DMA,
        ],
    )

    degp = deg_call(edge_d5, ones_k, zeros_p)

    hs1, dinv_b = pl.pallas_call(
        _tc1_body,
        out_shape=[
            jax.ShapeDtypeStruct((n, d_hid), jnp.float32),
            jax.ShapeDtypeStruct((n, d_hid), jnp.float32),
        ],
    )(x, W1, degp)

    p_part = prop_call(hs1, edge_p5, zeros_p)

    hs2, acc = pl.pallas_call(
        _tc_mid_body,
        out_shape=[
            jax.ShapeDtypeStruct((n, d_hid), jnp.float32),
            jax.ShapeDtypeStruct((n, d_out), jnp.float32),
        ],
    )(p_part, hs1, dinv_b, b1, W2, Wlin)

    q_part = prop_call(hs2, edge_p5, zeros_p)

    out = pl.pallas_call(
        _tc_fin_body,
        out_shape=jax.ShapeDtypeStruct((n, d_out), jnp.float32),
    )(q_part, hs2, dinv_b, b2, acc, Wlin, blin)

    return out
